# baseline (speedup 1.0000x reference)
import functools
import math

import jax
import jax.numpy as jnp
from jax import lax
from jax.experimental import pallas as pl
from jax.experimental.pallas import tpu as pltpu
from jax.experimental.pallas import tpu_sc as plsc

_V = 1_000_000
_K = 100
_KP = 128
_D = 64
_B = 4096
_N = 20
_R = _B * _N
_LOGK = math.log(_K)

_NC, _NS = 2, 16
_NW = _NC * _NS
_RPW = _R // _NW
_CH = 128
_NCHUNK = _RPW // _CH


def _sc_gather_body(idx_hbm, kk_hbm, embs_hbm, tgt_out, noise_out,
                    idx_v, kidx_v, rows_v, krows_v, sem):
    wid = lax.axis_index("s") * _NC + lax.axis_index("c")
    base = wid * _RPW
    pltpu.sync_copy(idx_hbm.at[wid], idx_v)
    for c in range(_NCHUNK):
        pltpu.async_copy(embs_hbm.at[idx_v.at[c]], rows_v, sem).wait()
        pltpu.sync_copy(rows_v, tgt_out.at[pl.ds(base + c * _CH, _CH)])

    @pl.when(wid == 0)
    def _noise():
        pltpu.sync_copy(kk_hbm, kidx_v)
        pltpu.async_copy(embs_hbm.at[kidx_v], krows_v, sem).wait()
        pltpu.sync_copy(krows_v, noise_out)


@functools.cache
def _sc_gather():
    return pl.kernel(
        _sc_gather_body,
        out_type=(
            jax.ShapeDtypeStruct((_R, _D), jnp.float32),
            jax.ShapeDtypeStruct((_KP, _D), jnp.float32),
        ),
        mesh=plsc.VectorSubcoreMesh(core_axis_name="c", subcore_axis_name="s"),
        scratch_types=[
            pltpu.VMEM((_NCHUNK, _CH), jnp.int32),
            pltpu.VMEM((_KP,), jnp.int32),
            pltpu.VMEM((_CH, _D), jnp.float32),
            pltpu.VMEM((_KP, _D), jnp.float32),
            pltpu.SemaphoreType.DMA,
        ],
        compiler_params=pltpu.CompilerParams(use_tc_tiling_on_sc=False),
    )


_CB = 1024
_NCB = _B // _CB


def _tc_body(x_ref, e_ref, nw_ref, out_ref):
    x = x_ref[0]
    e = e_ref[...]
    nw = nw_ref[...]
    et = e.T
    ts = jnp.sum(x * et, axis=0, keepdims=True)
    ns = lax.dot_general(nw, x, (((1,), (0,)), ((), ())),
                         preferred_element_type=jnp.float32)
    xt = _LOGK - ts
    lt = jnp.maximum(xt, 0.0) + jnp.log1p(jnp.exp(-jnp.abs(xt)))
    xn = ns - _LOGK
    ln = jnp.maximum(xn, 0.0) + jnp.log1p(jnp.exp(-jnp.abs(xn)))
    kmask = (lax.broadcasted_iota(jnp.int32, (_KP, 1), 0) < _K).astype(jnp.float32)
    part = (jnp.sum(ln * kmask) + jnp.sum(lt)) * (1.0 / _R)

    @pl.when((pl.program_id(0) == 0) & (pl.program_id(1) == 0))
    def _init():
        out_ref[...] = jnp.zeros_like(out_ref)

    out_ref[...] = out_ref[...] + part


_tc_loss = pl.pallas_call(
    _tc_body,
    grid=(_N, _NCB),
    in_specs=[
        pl.BlockSpec((1, _D, _CB), lambda n, c: (n, 0, c)),
        pl.BlockSpec((_CB, _D), lambda n, c: (n * _NCB + c, 0)),
        pl.BlockSpec((_KP, _D), lambda n, c: (0, 0)),
    ],
    out_specs=pl.BlockSpec((1, 1), lambda n, c: (0, 0)),
    out_shape=jax.ShapeDtypeStruct((1, 1), jnp.float32),
)


def kernel(target, input, embs):
    idx = target.T.astype(jnp.int32).reshape(_NW, _NCHUNK, _CH)
    kk = jax.random.randint(jax.random.key(123), (1, 1, _K), 0, _V)
    kk_pad = jnp.zeros((_KP,), jnp.int32).at[:_K].set(kk.reshape(-1).astype(jnp.int32))
    tgt_rows, noise_rows = _sc_gather()(idx, kk_pad, embs)
    x_t = jnp.transpose(input, (1, 2, 0))
    out = _tc_loss(x_t, tgt_rows, noise_rows)
    return out.reshape(())


# speedup vs baseline: 1.0218x; 1.0218x over previous
import functools
import math

import jax
import jax.numpy as jnp
from jax import lax
from jax.experimental import pallas as pl
from jax.experimental.pallas import tpu as pltpu
from jax.experimental.pallas import tpu_sc as plsc

_V = 1_000_000
_V2 = _V // 2
_K = 100
_KP = 128
_D = 64
_D2 = 128
_B = 4096
_N = 20
_R = _B * _N
_LOGK = math.log(_K)

_NC, _NS = 2, 16
_NW = _NC * _NS
_RPW = _R // _NW
_CH = 128
_NCHUNK = _RPW // _CH


def _sc_gather_body(idx_hbm, kk_hbm, embs_hbm, tgt_out, noise_out,
                    idx_v, kidx_v, rows_v, krows_v, sem):
    wid = lax.axis_index("s") * _NC + lax.axis_index("c")
    base = wid * _RPW
    pltpu.sync_copy(idx_hbm.at[wid], idx_v)
    for c in range(_NCHUNK):
        pltpu.async_copy(embs_hbm.at[idx_v.at[c]], rows_v, sem).wait()
        pltpu.sync_copy(rows_v, tgt_out.at[pl.ds(base + c * _CH, _CH)])

    @pl.when(wid == 0)
    def _noise():
        pltpu.sync_copy(kk_hbm, kidx_v)
        pltpu.async_copy(embs_hbm.at[kidx_v], krows_v, sem).wait()
        pltpu.sync_copy(krows_v, noise_out)


@functools.cache
def _sc_gather():
    return pl.kernel(
        _sc_gather_body,
        out_type=(
            jax.ShapeDtypeStruct((_R, _D2), jnp.float32),
            jax.ShapeDtypeStruct((_KP, _D2), jnp.float32),
        ),
        mesh=plsc.VectorSubcoreMesh(core_axis_name="c", subcore_axis_name="s"),
        scratch_types=[
            pltpu.VMEM((_NCHUNK, _CH), jnp.int32),
            pltpu.VMEM((_KP,), jnp.int32),
            pltpu.VMEM((_CH, _D2), jnp.float32),
            pltpu.VMEM((_KP, _D2), jnp.float32),
            pltpu.SemaphoreType.DMA,
        ],
    )


_CB = 1024
_NCB = _B // _CB


def _tc_body(x_ref, e_ref, par_ref, nw_ref, out_ref):
    x = x_ref[0]
    e = e_ref[...]
    nw = nw_ref[...]
    par = par_ref[0]
    et = e.T
    ts0 = jnp.sum(x * et[:_D], axis=0, keepdims=True)
    ts1 = jnp.sum(x * et[_D:], axis=0, keepdims=True)
    ts = jnp.where(par == 1, ts1, ts0)
    ns = lax.dot_general(nw, x, (((1,), (0,)), ((), ())),
                         preferred_element_type=jnp.float32)
    xt = _LOGK - ts
    lt = jnp.maximum(xt, 0.0) + jnp.log1p(jnp.exp(-jnp.abs(xt)))
    xn = ns - _LOGK
    ln = jnp.maximum(xn, 0.0) + jnp.log1p(jnp.exp(-jnp.abs(xn)))
    kmask = (lax.broadcasted_iota(jnp.int32, (_KP, 1), 0) < _K).astype(jnp.float32)
    part = (jnp.sum(ln * kmask) + jnp.sum(lt)) * (1.0 / _R)

    @pl.when((pl.program_id(0) == 0) & (pl.program_id(1) == 0))
    def _init():
        out_ref[...] = jnp.zeros_like(out_ref)

    out_ref[...] = out_ref[...] + part


_tc_loss = pl.pallas_call(
    _tc_body,
    grid=(_N, _NCB),
    in_specs=[
        pl.BlockSpec((1, _D, _CB), lambda n, c: (n, 0, c)),
        pl.BlockSpec((_CB, _D2), lambda n, c: (n * _NCB + c, 0)),
        pl.BlockSpec((1, 1, _CB), lambda n, c: (n, 0, c)),
        pl.BlockSpec((_KP, _D), lambda n, c: (0, 0)),
    ],
    out_specs=pl.BlockSpec((1, 1), lambda n, c: (0, 0)),
    out_shape=jax.ShapeDtypeStruct((1, 1), jnp.float32),
)


def kernel(target, input, embs):
    tgt_t = target.T.astype(jnp.int32)
    idx = (tgt_t >> 1).reshape(_NW, _NCHUNK, _CH)
    par = (tgt_t & 1).reshape(_N, 1, _B)
    kk = jax.random.randint(jax.random.key(123), (1, 1, _K), 0, _V)
    kk_pad = jnp.zeros((_KP,), jnp.int32).at[:_K].set(kk.reshape(-1).astype(jnp.int32))
    embs2 = embs.reshape(_V2, _D2)
    tgt_rows, krows = _sc_gather()(idx, kk_pad >> 1, embs2)
    nw = jnp.where((kk_pad & 1)[:, None] == 1, krows[:, _D:], krows[:, :_D])
    x_t = jnp.transpose(input, (1, 2, 0))
    out = _tc_loss(x_t, tgt_rows, par, nw)
    return out.reshape(())


# speedup vs baseline: 1.3411x; 1.3125x over previous
import functools
import math

import jax
import jax.numpy as jnp
from jax import lax
from jax.experimental import pallas as pl
from jax.experimental.pallas import tpu as pltpu
from jax.experimental.pallas import tpu_sc as plsc

_V = 1_000_000
_V2 = _V // 2
_K = 100
_KP = 128
_D = 64
_D2 = 128
_B = 4096
_N = 20
_R = _B * _N
_LOGK = math.log(_K)

_NC, _NS = 2, 16
_NW = _NC * _NS
_RPW = _R // _NW
_CH = 128
_NCHUNK = _RPW // _CH


def _sc_gather_body(idx_hbm, kk_hbm, embs_hbm, tgt_out, noise_out,
                    idx_v, kidx_v, rows_v, krows_v, sem):
    wid = lax.axis_index("s") * _NC + lax.axis_index("c")
    base = wid * _RPW
    pltpu.sync_copy(idx_hbm.at[wid], idx_v)
    for c in range(_NCHUNK):
        pltpu.async_copy(embs_hbm.at[idx_v.at[c]], rows_v, sem).wait()
        pltpu.sync_copy(rows_v, tgt_out.at[pl.ds(base + c * _CH, _CH)])

    @pl.when(wid == 0)
    def _noise():
        pltpu.sync_copy(kk_hbm, kidx_v)
        pltpu.async_copy(embs_hbm.at[kidx_v], krows_v, sem).wait()
        pltpu.sync_copy(krows_v, noise_out)


@functools.cache
def _sc_gather():
    return pl.kernel(
        _sc_gather_body,
        out_type=(
            jax.ShapeDtypeStruct((_R, _D2), jnp.float32),
            jax.ShapeDtypeStruct((_KP, _D2), jnp.float32),
        ),
        mesh=plsc.VectorSubcoreMesh(core_axis_name="c", subcore_axis_name="s"),
        scratch_types=[
            pltpu.VMEM((_NCHUNK, _CH), jnp.int32),
            pltpu.VMEM((_KP,), jnp.int32),
            pltpu.VMEM((_CH, _D2), jnp.float32),
            pltpu.VMEM((_KP, _D2), jnp.float32),
            pltpu.SemaphoreType.DMA,
        ],
    )


_PW = 4096
_PG = (_V + _PW - 1) // _PW


def _pack_body(xt_ref, out_ref):
    x = xt_ref[...]
    xt = x.T
    nxt = pltpu.roll(xt, _PW - 1, 0)
    y = jnp.concatenate([xt, nxt], axis=1)
    out_ref[...] = y.reshape(_PW // 2, 2, _D2)[:, 0, :]


_pack = pl.pallas_call(
    _pack_body,
    grid=(_PG,),
    in_specs=[pl.BlockSpec((_D, _PW), lambda c: (0, c))],
    out_specs=pl.BlockSpec((_PW // 2, _D2), lambda c: (c, 0)),
    out_shape=jax.ShapeDtypeStruct((_V2, _D2), jnp.float32),
)


_CB = 1024
_NCB = _B // _CB


def _tc_body(x_ref, e_ref, par_ref, nw_ref, out_ref):
    x = x_ref[0]
    e = e_ref[...]
    nw = nw_ref[...]
    par = par_ref[0]
    et = e.T
    ts0 = jnp.sum(x * et[:_D], axis=0, keepdims=True)
    ts1 = jnp.sum(x * et[_D:], axis=0, keepdims=True)
    ts = jnp.where(par == 1, ts1, ts0)
    ns = lax.dot_general(nw, x, (((1,), (0,)), ((), ())),
                         preferred_element_type=jnp.float32)
    xt = _LOGK - ts
    lt = jnp.maximum(xt, 0.0) + jnp.log1p(jnp.exp(-jnp.abs(xt)))
    xn = ns - _LOGK
    ln = jnp.maximum(xn, 0.0) + jnp.log1p(jnp.exp(-jnp.abs(xn)))
    kmask = (lax.broadcasted_iota(jnp.int32, (_KP, 1), 0) < _K).astype(jnp.float32)
    part = (jnp.sum(ln * kmask) + jnp.sum(lt)) * (1.0 / _R)

    @pl.when((pl.program_id(0) == 0) & (pl.program_id(1) == 0))
    def _init():
        out_ref[...] = jnp.zeros_like(out_ref)

    out_ref[...] = out_ref[...] + part


_tc_loss = pl.pallas_call(
    _tc_body,
    grid=(_N, _NCB),
    in_specs=[
        pl.BlockSpec((1, _D, _CB), lambda n, c: (n, 0, c)),
        pl.BlockSpec((_CB, _D2), lambda n, c: (n * _NCB + c, 0)),
        pl.BlockSpec((1, 1, _CB), lambda n, c: (n, 0, c)),
        pl.BlockSpec((_KP, _D), lambda n, c: (0, 0)),
    ],
    out_specs=pl.BlockSpec((1, 1), lambda n, c: (0, 0)),
    out_shape=jax.ShapeDtypeStruct((1, 1), jnp.float32),
)


def kernel(target, input, embs):
    tgt_t = target.T.astype(jnp.int32)
    idx = (tgt_t >> 1).reshape(_NW, _NCHUNK, _CH)
    par = (tgt_t & 1).reshape(_N, 1, _B)
    kk = jax.random.randint(jax.random.key(123), (1, 1, _K), 0, _V)
    kk_pad = jnp.zeros((_KP,), jnp.int32).at[:_K].set(kk.reshape(-1).astype(jnp.int32))
    embs2 = _pack(embs.T)
    tgt_rows, krows = _sc_gather()(idx, kk_pad >> 1, embs2)
    nw = jnp.where((kk_pad & 1)[:, None] == 1, krows[:, _D:], krows[:, :_D])
    x_t = jnp.transpose(input, (1, 2, 0))
    out = _tc_loss(x_t, tgt_rows, par, nw)
    return out.reshape(())


# speedup vs baseline: 1.6620x; 1.2393x over previous
import functools
import math

import jax
import jax.numpy as jnp
from jax import lax
from jax.experimental import pallas as pl
from jax.experimental.pallas import tpu as pltpu
from jax.experimental.pallas import tpu_sc as plsc

_V = 1_000_000
_V2 = _V // 2
_K = 100
_KP = 128
_D = 64
_D2 = 128
_B = 4096
_N = 20
_R = _B * _N
_LOGK = math.log(_K)

_NC, _NS = 2, 16
_NW = _NC * _NS
_RPW = _R // _NW
_CH = 128
_NCHUNK = _RPW // _CH


def _sc_gather_body(idx_hbm, kk_hbm, embs_hbm, tgt_out, noise_out,
                    idx_v, kidx_v, rows_v, krows_v, sem):
    wid = lax.axis_index("s") * _NC + lax.axis_index("c")
    base = wid * _RPW
    pltpu.sync_copy(idx_hbm.at[wid], idx_v)
    for c in range(_NCHUNK):
        pltpu.async_copy(embs_hbm.at[idx_v.at[c]], rows_v, sem).wait()
        pltpu.sync_copy(rows_v, tgt_out.at[pl.ds(base + c * _CH, _CH)])

    @pl.when(wid == 0)
    def _noise():
        pltpu.sync_copy(kk_hbm, kidx_v)
        pltpu.async_copy(embs_hbm.at[kidx_v], krows_v, sem).wait()
        pltpu.sync_copy(krows_v, noise_out)


@functools.cache
def _sc_gather():
    return pl.kernel(
        _sc_gather_body,
        out_type=(
            jax.ShapeDtypeStruct((_R, _D2), jnp.float32),
            jax.ShapeDtypeStruct((_KP, _D2), jnp.float32),
        ),
        mesh=plsc.VectorSubcoreMesh(core_axis_name="c", subcore_axis_name="s"),
        scratch_types=[
            pltpu.VMEM((_NCHUNK, _CH), jnp.int32),
            pltpu.VMEM((_KP,), jnp.int32),
            pltpu.VMEM((_CH, _D2), jnp.float32),
            pltpu.VMEM((_KP, _D2), jnp.float32),
            pltpu.SemaphoreType.DMA,
        ],
    )


_PW = 4096
_PG = (_V + _PW - 1) // _PW


_PH = _PW // 2
_TROWS = _PG * _PH


def _pack_body(xt_ref, out_ref):
    x = xt_ref[...]
    il = lax.broadcasted_iota(jnp.int32, (_D, _D2), 1)
    ir = lax.broadcasted_iota(jnp.int32, (_D, _D2), 0)
    p1 = (il == ir).astype(jnp.float32)
    p2 = (il == ir + _D).astype(jnp.float32)
    cd = (((0,), (0,)), ((), ()))
    out_ref[...] = (
        lax.dot_general(x[:, :_PH], p1, cd, preferred_element_type=jnp.float32)
        + lax.dot_general(x[:, _PH:], p2, cd, preferred_element_type=jnp.float32)
    )


_pack = pl.pallas_call(
    _pack_body,
    grid=(_PG,),
    in_specs=[pl.BlockSpec((_D, _PW), lambda c: (0, c))],
    out_specs=pl.BlockSpec((_PH, _D2), lambda c: (c, 0)),
    out_shape=jax.ShapeDtypeStruct((_TROWS, _D2), jnp.float32),
)


_CB = 1024
_NCB = _B // _CB


def _tc_body(x_ref, e_ref, par_ref, nw_ref, out_ref):
    x = x_ref[0]
    e = e_ref[...]
    nw = nw_ref[...]
    par = par_ref[0]
    et = e.T
    ts0 = jnp.sum(x * et[:_D], axis=0, keepdims=True)
    ts1 = jnp.sum(x * et[_D:], axis=0, keepdims=True)
    ts = jnp.where(par == 1, ts1, ts0)
    ns = lax.dot_general(nw, x, (((1,), (0,)), ((), ())),
                         preferred_element_type=jnp.float32)
    xt = _LOGK - ts
    lt = jnp.maximum(xt, 0.0) + jnp.log1p(jnp.exp(-jnp.abs(xt)))
    xn = ns - _LOGK
    ln = jnp.maximum(xn, 0.0) + jnp.log1p(jnp.exp(-jnp.abs(xn)))
    kmask = (lax.broadcasted_iota(jnp.int32, (_KP, 1), 0) < _K).astype(jnp.float32)
    part = (jnp.sum(ln * kmask) + jnp.sum(lt)) * (1.0 / _R)

    @pl.when((pl.program_id(0) == 0) & (pl.program_id(1) == 0))
    def _init():
        out_ref[...] = jnp.zeros_like(out_ref)

    out_ref[...] = out_ref[...] + part


_tc_loss = pl.pallas_call(
    _tc_body,
    grid=(_N, _NCB),
    in_specs=[
        pl.BlockSpec((1, _D, _CB), lambda n, c: (n, 0, c)),
        pl.BlockSpec((_CB, _D2), lambda n, c: (n * _NCB + c, 0)),
        pl.BlockSpec((1, 1, _CB), lambda n, c: (n, 0, c)),
        pl.BlockSpec((_KP, _D), lambda n, c: (0, 0)),
    ],
    out_specs=pl.BlockSpec((1, 1), lambda n, c: (0, 0)),
    out_shape=jax.ShapeDtypeStruct((1, 1), jnp.float32),
)


def kernel(target, input, embs):
    tgt_t = target.T.astype(jnp.int32)
    row = ((tgt_t >> 12) << 11) + (tgt_t & (_PH - 1))
    idx = row.reshape(_NW, _NCHUNK, _CH)
    par = ((tgt_t >> 11) & 1).reshape(_N, 1, _B)
    kk = jax.random.randint(jax.random.key(123), (1, 1, _K), 0, _V)
    kk_pad = jnp.zeros((_KP,), jnp.int32).at[:_K].set(kk.reshape(-1).astype(jnp.int32))
    krow = ((kk_pad >> 12) << 11) + (kk_pad & (_PH - 1))
    embs2 = _pack(embs.T)
    tgt_rows, krows = _sc_gather()(idx, krow, embs2)
    nw = jnp.where((((kk_pad >> 11) & 1) == 1)[:, None], krows[:, _D:], krows[:, :_D])
    x_t = jnp.transpose(input, (1, 2, 0))
    out = _tc_loss(x_t, tgt_rows, par, nw)
    return out.reshape(())


# speedup vs baseline: 1.9894x; 1.1970x over previous
import functools
import math

import jax
import jax.numpy as jnp
from jax import lax
from jax.experimental import pallas as pl
from jax.experimental.pallas import tpu as pltpu
from jax.experimental.pallas import tpu_sc as plsc

_V = 1_000_000
_V2 = _V // 2
_K = 100
_KP = 128
_D = 64
_D2 = 128
_B = 4096
_N = 20
_R = _B * _N
_LOGK = math.log(_K)

_NC, _NS = 2, 16
_NW = _NC * _NS
_RPW = _R // _NW
_CH = 128
_NCHUNK = _RPW // _CH


def _sc_gather_body(idx_hbm, kk_hbm, embs_hbm, tgt_out, noise_out,
                    idx_v, kidx_v, rows_v, krows_v, sem):
    wid = lax.axis_index("s") * _NC + lax.axis_index("c")
    base = wid * _RPW
    pltpu.sync_copy(idx_hbm.at[wid], idx_v)
    for c in range(_NCHUNK):
        pltpu.async_copy(embs_hbm.at[idx_v.at[c]], rows_v, sem).wait()
        pltpu.sync_copy(rows_v, tgt_out.at[pl.ds(base + c * _CH, _CH)])

    @pl.when(wid == 0)
    def _noise():
        pltpu.sync_copy(kk_hbm, kidx_v)
        pltpu.async_copy(embs_hbm.at[kidx_v], krows_v, sem).wait()
        pltpu.sync_copy(krows_v, noise_out)


@functools.cache
def _sc_gather():
    return pl.kernel(
        _sc_gather_body,
        out_type=(
            jax.ShapeDtypeStruct((_R, _D2), jnp.float32),
            jax.ShapeDtypeStruct((_KP, _D2), jnp.float32),
        ),
        mesh=plsc.VectorSubcoreMesh(core_axis_name="c", subcore_axis_name="s"),
        scratch_types=[
            pltpu.VMEM((_NCHUNK, _CH), jnp.int32),
            pltpu.VMEM((_KP,), jnp.int32),
            pltpu.VMEM((_CH, _D2), jnp.float32),
            pltpu.VMEM((_KP, _D2), jnp.float32),
            pltpu.SemaphoreType.DMA,
        ],
    )


_PW = 8192
_PG = (_V + _PW - 1) // _PW


_PH = _PW // 2
_TROWS = _PG * _PH


def _pack_body(xt_ref, out_ref):
    x = xt_ref[...]
    il = lax.broadcasted_iota(jnp.int32, (_D, _D2), 1)
    ir = lax.broadcasted_iota(jnp.int32, (_D, _D2), 0)
    p1 = (il == ir).astype(jnp.float32)
    p2 = (il == ir + _D).astype(jnp.float32)
    cd = (((0,), (0,)), ((), ()))
    out_ref[...] = (
        lax.dot_general(x[:, :_PH], p1, cd, preferred_element_type=jnp.float32)
        + lax.dot_general(x[:, _PH:], p2, cd, preferred_element_type=jnp.float32)
    )


_pack = pl.pallas_call(
    _pack_body,
    grid=(_PG,),
    in_specs=[pl.BlockSpec((_D, _PW), lambda c: (0, c))],
    out_specs=pl.BlockSpec((_PH, _D2), lambda c: (c, 0)),
    out_shape=jax.ShapeDtypeStruct((_TROWS, _D2), jnp.float32),
)


_CB = 1024
_NCB = _B // _CB


def _tc_body(x_ref, e_ref, par_ref, nw_ref, out_ref):
    x = x_ref[0]
    e = e_ref[...]
    nw = nw_ref[...]
    par = par_ref[0]
    et = e.T
    ts0 = jnp.sum(x * et[:_D], axis=0, keepdims=True)
    ts1 = jnp.sum(x * et[_D:], axis=0, keepdims=True)
    ts = jnp.where(par == 1, ts1, ts0)
    ns = lax.dot_general(nw, x, (((1,), (0,)), ((), ())),
                         preferred_element_type=jnp.float32)
    xt = _LOGK - ts
    lt = jnp.maximum(xt, 0.0) + jnp.log1p(jnp.exp(-jnp.abs(xt)))
    xn = ns - _LOGK
    ln = jnp.maximum(xn, 0.0) + jnp.log1p(jnp.exp(-jnp.abs(xn)))
    kmask = (lax.broadcasted_iota(jnp.int32, (_KP, 1), 0) < _K).astype(jnp.float32)
    part = (jnp.sum(ln * kmask) + jnp.sum(lt)) * (1.0 / _R)

    @pl.when((pl.program_id(0) == 0) & (pl.program_id(1) == 0))
    def _init():
        out_ref[...] = jnp.zeros_like(out_ref)

    out_ref[...] = out_ref[...] + part


_tc_loss = pl.pallas_call(
    _tc_body,
    grid=(_N, _NCB),
    in_specs=[
        pl.BlockSpec((1, _D, _CB), lambda n, c: (n, 0, c)),
        pl.BlockSpec((_CB, _D2), lambda n, c: (n * _NCB + c, 0)),
        pl.BlockSpec((1, 1, _CB), lambda n, c: (n, 0, c)),
        pl.BlockSpec((_KP, _D), lambda n, c: (0, 0)),
    ],
    out_specs=pl.BlockSpec((1, 1), lambda n, c: (0, 0)),
    out_shape=jax.ShapeDtypeStruct((1, 1), jnp.float32),
)


def kernel(target, input, embs):
    tgt_t = target.T.astype(jnp.int32)
    row = ((tgt_t >> 13) << 12) + (tgt_t & (_PH - 1))
    idx = row.reshape(_NW, _NCHUNK, _CH)
    par = ((tgt_t >> 12) & 1).reshape(_N, 1, _B)
    kk = jax.random.randint(jax.random.key(123), (1, 1, _K), 0, _V)
    kk_pad = jnp.zeros((_KP,), jnp.int32).at[:_K].set(kk.reshape(-1).astype(jnp.int32))
    krow = ((kk_pad >> 13) << 12) + (kk_pad & (_PH - 1))
    embs2 = _pack(embs.T)
    tgt_rows, krows = _sc_gather()(idx, krow, embs2)
    nw = jnp.where((((kk_pad >> 12) & 1) == 1)[:, None], krows[:, _D:], krows[:, :_D])
    x_t = jnp.transpose(input, (1, 2, 0))
    out = _tc_loss(x_t, tgt_rows, par, nw)
    return out.reshape(())


# speedup vs baseline: 2.1987x; 1.1052x over previous
import functools
import math

import jax
import jax.numpy as jnp
from jax import lax
from jax.experimental import pallas as pl
from jax.experimental.pallas import tpu as pltpu
from jax.experimental.pallas import tpu_sc as plsc

_V = 1_000_000
_V2 = _V // 2
_K = 100
_KP = 128
_D = 64
_D2 = 128
_B = 4096
_N = 20
_R = _B * _N
_LOGK = math.log(_K)

_NC, _NS = 2, 16
_NW = _NC * _NS
_RPW = _R // _NW
_CH = 128
_NCHUNK = _RPW // _CH


def _sc_gather_body(idx_hbm, kk_hbm, embs_hbm, tgt_out, noise_out,
                    idx_v, kidx_v, rows_v, krows_v, sem):
    wid = lax.axis_index("s") * _NC + lax.axis_index("c")
    base = wid * _RPW
    pltpu.sync_copy(idx_hbm.at[wid], idx_v)
    for c in range(_NCHUNK):
        pltpu.async_copy(embs_hbm.at[idx_v.at[c]], rows_v, sem).wait()
        pltpu.sync_copy(rows_v, tgt_out.at[pl.ds(base + c * _CH, _CH)])

    @pl.when(wid == 0)
    def _noise():
        pltpu.sync_copy(kk_hbm, kidx_v)
        pltpu.async_copy(embs_hbm.at[kidx_v], krows_v, sem).wait()
        pltpu.sync_copy(krows_v, noise_out)


@functools.cache
def _sc_gather():
    return pl.kernel(
        _sc_gather_body,
        out_type=(
            jax.ShapeDtypeStruct((_R, _D2), jnp.float32),
            jax.ShapeDtypeStruct((_KP, _D2), jnp.float32),
        ),
        mesh=plsc.VectorSubcoreMesh(core_axis_name="c", subcore_axis_name="s"),
        scratch_types=[
            pltpu.VMEM((_NCHUNK, _CH), jnp.int32),
            pltpu.VMEM((_KP,), jnp.int32),
            pltpu.VMEM((_CH, _D2), jnp.float32),
            pltpu.VMEM((_KP, _D2), jnp.float32),
            pltpu.SemaphoreType.DMA,
        ],
    )


_PW = 16384
_PG = (_V + _PW - 1) // _PW


_PH = _PW // 2
_TROWS = _PG * _PH


def _pack_body(xt_ref, out_ref):
    x = xt_ref[...]
    il = lax.broadcasted_iota(jnp.int32, (_D, _D2), 1)
    ir = lax.broadcasted_iota(jnp.int32, (_D, _D2), 0)
    p1 = (il == ir).astype(jnp.float32)
    p2 = (il == ir + _D).astype(jnp.float32)
    cd = (((0,), (0,)), ((), ()))
    out_ref[...] = (
        lax.dot_general(x[:, :_PH], p1, cd, preferred_element_type=jnp.float32)
        + lax.dot_general(x[:, _PH:], p2, cd, preferred_element_type=jnp.float32)
    )


_pack = pl.pallas_call(
    _pack_body,
    grid=(_PG,),
    in_specs=[pl.BlockSpec((_D, _PW), lambda c: (0, c))],
    out_specs=pl.BlockSpec((_PH, _D2), lambda c: (c, 0)),
    out_shape=jax.ShapeDtypeStruct((_TROWS, _D2), jnp.float32),
)


_CB = 1024
_NCB = _B // _CB


def _tc_body(x_ref, e_ref, par_ref, nw_ref, out_ref):
    x = x_ref[0]
    e = e_ref[...]
    nw = nw_ref[...]
    par = par_ref[0]
    et = e.T
    ts0 = jnp.sum(x * et[:_D], axis=0, keepdims=True)
    ts1 = jnp.sum(x * et[_D:], axis=0, keepdims=True)
    ts = jnp.where(par == 1, ts1, ts0)
    ns = lax.dot_general(nw, x, (((1,), (0,)), ((), ())),
                         preferred_element_type=jnp.float32)
    xt = _LOGK - ts
    lt = jnp.maximum(xt, 0.0) + jnp.log1p(jnp.exp(-jnp.abs(xt)))
    xn = ns - _LOGK
    ln = jnp.maximum(xn, 0.0) + jnp.log1p(jnp.exp(-jnp.abs(xn)))
    kmask = (lax.broadcasted_iota(jnp.int32, (_KP, 1), 0) < _K).astype(jnp.float32)
    part = (jnp.sum(ln * kmask) + jnp.sum(lt)) * (1.0 / _R)

    @pl.when((pl.program_id(0) == 0) & (pl.program_id(1) == 0))
    def _init():
        out_ref[...] = jnp.zeros_like(out_ref)

    out_ref[...] = out_ref[...] + part


_tc_loss = pl.pallas_call(
    _tc_body,
    grid=(_N, _NCB),
    in_specs=[
        pl.BlockSpec((1, _D, _CB), lambda n, c: (n, 0, c)),
        pl.BlockSpec((_CB, _D2), lambda n, c: (n * _NCB + c, 0)),
        pl.BlockSpec((1, 1, _CB), lambda n, c: (n, 0, c)),
        pl.BlockSpec((_KP, _D), lambda n, c: (0, 0)),
    ],
    out_specs=pl.BlockSpec((1, 1), lambda n, c: (0, 0)),
    out_shape=jax.ShapeDtypeStruct((1, 1), jnp.float32),
)


def kernel(target, input, embs):
    tgt_t = target.T.astype(jnp.int32)
    row = ((tgt_t >> 14) << 13) + (tgt_t & (_PH - 1))
    idx = row.reshape(_NW, _NCHUNK, _CH)
    par = ((tgt_t >> 13) & 1).reshape(_N, 1, _B)
    kk = jax.random.randint(jax.random.key(123), (1, 1, _K), 0, _V)
    kk_pad = jnp.zeros((_KP,), jnp.int32).at[:_K].set(kk.reshape(-1).astype(jnp.int32))
    krow = ((kk_pad >> 14) << 13) + (kk_pad & (_PH - 1))
    embs2 = _pack(embs.T)
    tgt_rows, krows = _sc_gather()(idx, krow, embs2)
    nw = jnp.where((((kk_pad >> 13) & 1) == 1)[:, None], krows[:, _D:], krows[:, :_D])
    x_t = jnp.transpose(input, (1, 2, 0))
    out = _tc_loss(x_t, tgt_rows, par, nw)
    return out.reshape(())


# speedup vs baseline: 2.3267x; 1.0582x over previous
import functools
import math

import jax
import jax.numpy as jnp
from jax import lax
from jax.experimental import pallas as pl
from jax.experimental.pallas import tpu as pltpu
from jax.experimental.pallas import tpu_sc as plsc

_V = 1_000_000
_V2 = _V // 2
_K = 100
_KP = 128
_D = 64
_D2 = 128
_B = 4096
_N = 20
_R = _B * _N
_LOGK = math.log(_K)

_NC, _NS = 2, 16
_NW = _NC * _NS
_RPW = _R // _NW
_CH = 128
_NCHUNK = _RPW // _CH


def _sc_gather_body(idx_hbm, kk_hbm, embs_hbm, tgt_out, noise_out,
                    idx_v, kidx_v, rows_v, krows_v, sem):
    wid = lax.axis_index("s") * _NC + lax.axis_index("c")
    base = wid * _RPW
    pltpu.sync_copy(idx_hbm.at[wid], idx_v)
    for c in range(_NCHUNK):
        pltpu.async_copy(embs_hbm.at[idx_v.at[c]], rows_v, sem).wait()
        pltpu.sync_copy(rows_v, tgt_out.at[pl.ds(base + c * _CH, _CH)])

    @pl.when(wid == 0)
    def _noise():
        pltpu.sync_copy(kk_hbm, kidx_v)
        pltpu.async_copy(embs_hbm.at[kidx_v], krows_v, sem).wait()
        pltpu.sync_copy(krows_v, noise_out)


@functools.cache
def _sc_gather():
    return pl.kernel(
        _sc_gather_body,
        out_type=(
            jax.ShapeDtypeStruct((_R, _D2), jnp.float32),
            jax.ShapeDtypeStruct((_KP, _D2), jnp.float32),
        ),
        mesh=plsc.VectorSubcoreMesh(core_axis_name="c", subcore_axis_name="s"),
        scratch_types=[
            pltpu.VMEM((_NCHUNK, _CH), jnp.int32),
            pltpu.VMEM((_KP,), jnp.int32),
            pltpu.VMEM((_CH, _D2), jnp.float32),
            pltpu.VMEM((_KP, _D2), jnp.float32),
            pltpu.SemaphoreType.DMA,
        ],
    )


_PW = 32768
_PG = (_V + _PW - 1) // _PW


_PH = _PW // 2
_TROWS = _PG * _PH


def _pack_body(xt_ref, out_ref):
    x = xt_ref[...]
    il = lax.broadcasted_iota(jnp.int32, (_D, _D2), 1)
    ir = lax.broadcasted_iota(jnp.int32, (_D, _D2), 0)
    p1 = (il == ir).astype(jnp.float32)
    p2 = (il == ir + _D).astype(jnp.float32)
    cd = (((0,), (0,)), ((), ()))
    out_ref[...] = (
        lax.dot_general(x[:, :_PH], p1, cd, preferred_element_type=jnp.float32)
        + lax.dot_general(x[:, _PH:], p2, cd, preferred_element_type=jnp.float32)
    )


_pack = pl.pallas_call(
    _pack_body,
    grid=(_PG,),
    in_specs=[pl.BlockSpec((_D, _PW), lambda c: (0, c))],
    out_specs=pl.BlockSpec((_PH, _D2), lambda c: (c, 0)),
    out_shape=jax.ShapeDtypeStruct((_TROWS, _D2), jnp.float32),
)


_CB = 1024
_NCB = _B // _CB


def _tc_body(x_ref, e_ref, par_ref, nw_ref, out_ref):
    x = x_ref[0]
    e = e_ref[...]
    nw = nw_ref[...]
    par = par_ref[0]
    et = e.T
    ts0 = jnp.sum(x * et[:_D], axis=0, keepdims=True)
    ts1 = jnp.sum(x * et[_D:], axis=0, keepdims=True)
    ts = jnp.where(par == 1, ts1, ts0)
    ns = lax.dot_general(nw, x, (((1,), (0,)), ((), ())),
                         preferred_element_type=jnp.float32)
    xt = _LOGK - ts
    lt = jnp.maximum(xt, 0.0) + jnp.log1p(jnp.exp(-jnp.abs(xt)))
    xn = ns - _LOGK
    ln = jnp.maximum(xn, 0.0) + jnp.log1p(jnp.exp(-jnp.abs(xn)))
    kmask = (lax.broadcasted_iota(jnp.int32, (_KP, 1), 0) < _K).astype(jnp.float32)
    part = (jnp.sum(ln * kmask) + jnp.sum(lt)) * (1.0 / _R)

    @pl.when((pl.program_id(0) == 0) & (pl.program_id(1) == 0))
    def _init():
        out_ref[...] = jnp.zeros_like(out_ref)

    out_ref[...] = out_ref[...] + part


_tc_loss = pl.pallas_call(
    _tc_body,
    grid=(_N, _NCB),
    in_specs=[
        pl.BlockSpec((1, _D, _CB), lambda n, c: (n, 0, c)),
        pl.BlockSpec((_CB, _D2), lambda n, c: (n * _NCB + c, 0)),
        pl.BlockSpec((1, 1, _CB), lambda n, c: (n, 0, c)),
        pl.BlockSpec((_KP, _D), lambda n, c: (0, 0)),
    ],
    out_specs=pl.BlockSpec((1, 1), lambda n, c: (0, 0)),
    out_shape=jax.ShapeDtypeStruct((1, 1), jnp.float32),
)


def kernel(target, input, embs):
    tgt_t = target.T.astype(jnp.int32)
    row = ((tgt_t >> 15) << 14) + (tgt_t & (_PH - 1))
    idx = row.reshape(_NW, _NCHUNK, _CH)
    par = ((tgt_t >> 14) & 1).reshape(_N, 1, _B)
    kk = jax.random.randint(jax.random.key(123), (1, 1, _K), 0, _V)
    kk_pad = jnp.zeros((_KP,), jnp.int32).at[:_K].set(kk.reshape(-1).astype(jnp.int32))
    krow = ((kk_pad >> 15) << 14) + (kk_pad & (_PH - 1))
    embs2 = _pack(embs.T)
    tgt_rows, krows = _sc_gather()(idx, krow, embs2)
    nw = jnp.where((((kk_pad >> 14) & 1) == 1)[:, None], krows[:, _D:], krows[:, :_D])
    x_t = jnp.transpose(input, (1, 2, 0))
    out = _tc_loss(x_t, tgt_rows, par, nw)
    return out.reshape(())
